# baseline (speedup 1.0000x reference)
import functools

import jax
import jax.numpy as jnp
from jax import lax
from jax.experimental import pallas as pl
from jax.experimental.pallas import tpu as pltpu
from jax.experimental.pallas import tpu_sc as plsc

N = 10000
E = 160000
NW = 32
EPT = 5120
CH = 128
NCH = EPT // CH
EPAD = NW * EPT
NP = 10112
RPT = NP // 16
DUMMY = N
RB = 1264
GN = 8
EB = EPAD // 8

_f32 = jnp.float32
_mesh = plsc.VectorSubcoreMesh(core_axis_name="c", subcore_axis_name="s")


def _wid(c, s):
    return s * 2 + c


def _zero_stripe(zeros_hbm, accum, s):
    pltpu.sync_copy(zeros_hbm.at[pl.ds(s * RPT, RPT)],
                    accum.at[pl.ds(s * RPT, RPT)])


def _dump_stripe(accum, out_hbm, c, s):
    pltpu.sync_copy(accum.at[pl.ds(s * RPT, RPT)],
                    out_hbm.at[c, pl.ds(s * RPT, RPT)])


def _deg_body(idxs, ev0, ev1, zeros128, out, srcv, dstv, vb0, vb1, acc):
    c = lax.axis_index("c")
    s = lax.axis_index("s")
    w = _wid(c, s)
    _zero_stripe(zeros128, acc, s)
    pltpu.sync_copy(ev0, vb0)
    pltpu.sync_copy(ev1, vb1)
    pltpu.sync_copy(idxs.at[w, 0], srcv)
    pltpu.sync_copy(idxs.at[w, 1], dstv)
    plsc.subcore_barrier()

    def step(j, carry):
        pltpu.sync_copy(vb0, acc.at[srcv.at[j]], add=True)
        pltpu.sync_copy(vb1, acc.at[dstv.at[j]], add=True)
        return carry

    lax.fori_loop(0, NCH, step, 0)
    plsc.subcore_barrier()
    _dump_stripe(acc, out, c, s)


def _sc_degrees(idxs, ev0, ev1, zeros128):
    fn = pl.kernel(
        _deg_body,
        out_type=jax.ShapeDtypeStruct((2, NP, 128), _f32),
        mesh=_mesh,
        scratch_types=[
            pltpu.VMEM((NCH, CH), jnp.int32),
            pltpu.VMEM((NCH, CH), jnp.int32),
            pltpu.VMEM((CH, 128), _f32),
            pltpu.VMEM((CH, 128), _f32),
            pltpu.VMEM_SHARED((NP, 128), _f32),
        ],
    )
    return fn(idxs, ev0, ev1, zeros128)


def _agg_body(table, idxs, zeros128, out, srcv, dstv, gb0, gb1, accum,
              sem0, sem1, ssem0, ssem1):
    c = lax.axis_index("c")
    s = lax.axis_index("s")
    w = _wid(c, s)
    _zero_stripe(zeros128, accum, s)
    pltpu.sync_copy(idxs.at[w, 0], srcv)
    pltpu.sync_copy(idxs.at[w, 1], dstv)
    plsc.subcore_barrier()

    pltpu.async_copy(table.at[srcv.at[0]], gb0, sem0)
    pltpu.async_copy(table.at[srcv.at[1]], gb1, sem1)

    def step(j2, carry):
        a = 2 * j2
        b = a + 1
        pltpu.make_async_copy(table.at[srcv.at[a]], gb0, sem0).wait()
        pltpu.async_copy(gb0, accum.at[dstv.at[a]], ssem0, add=True)
        pltpu.make_async_copy(table.at[srcv.at[b]], gb1, sem1).wait()
        pltpu.async_copy(gb1, accum.at[dstv.at[b]], ssem1, add=True)
        pltpu.make_async_copy(gb0, accum.at[dstv.at[a]], ssem0).wait()

        @pl.when(j2 < NCH // 2 - 1)
        def _():
            pltpu.async_copy(table.at[srcv.at[a + 2]], gb0, sem0)

        pltpu.make_async_copy(gb1, accum.at[dstv.at[b]], ssem1).wait()

        @pl.when(j2 < NCH // 2 - 1)
        def _():
            pltpu.async_copy(table.at[srcv.at[b + 2]], gb1, sem1)

        return carry

    lax.fori_loop(0, NCH // 2, step, 0)
    plsc.subcore_barrier()
    _dump_stripe(accum, out, c, s)


def _sc_agg(table, idxs, zeros128):
    fn = pl.kernel(
        _agg_body,
        out_type=jax.ShapeDtypeStruct((2, NP, 128), _f32),
        mesh=_mesh,
        scratch_types=[
            pltpu.VMEM((NCH, CH), jnp.int32),
            pltpu.VMEM((NCH, CH), jnp.int32),
            pltpu.VMEM((CH, 128), _f32),
            pltpu.VMEM((CH, 128), _f32),
            pltpu.VMEM_SHARED((NP, 128), _f32),
            pltpu.SemaphoreType.DMA,
            pltpu.SemaphoreType.DMA,
            pltpu.SemaphoreType.DMA,
            pltpu.SemaphoreType.DMA,
        ],
    )
    return fn(table, idxs, zeros128)


def _edge_body(elq, erq, eep, idxs, zeros128, w_out, den_out,
               srcv, dstv, elb, erb, eeb, wb, dacc, sem, sem2):
    c = lax.axis_index("c")
    s = lax.axis_index("s")
    w = _wid(c, s)
    _zero_stripe(zeros128, dacc, s)
    pltpu.sync_copy(idxs.at[w, 0], srcv)
    pltpu.sync_copy(idxs.at[w, 1], dstv)
    plsc.subcore_barrier()

    def step(j, carry):
        wrow = w * (EPT // 8) + j * (CH // 8)
        cp0 = pltpu.async_copy(elq.at[srcv.at[j]], elb, sem)
        cp1 = pltpu.async_copy(erq.at[dstv.at[j]], erb, sem2)
        pltpu.sync_copy(eep.at[pl.ds(wrow, CH // 8)], eeb)
        cp0.wait()
        cp1.wait()

        def inner(r, carry2):
            for e8 in range(8):
                e = r * 8 + e8
                v = (elb[e, pl.ds(0, 16)] + erb[e, pl.ds(0, 16)]
                     + eeb[r, pl.ds(e8 * 16, 16)])
                v = jnp.where(v > 0, v, 0.2 * v)
                v = jnp.exp(v)
                wb[r, pl.ds(e8 * 16, 16)] = v
                elb[e, pl.ds(0, 16)] = v
            return carry2

        lax.fori_loop(0, CH // 8, inner, 0)
        pltpu.sync_copy(wb, w_out.at[pl.ds(wrow, CH // 8)])
        pltpu.sync_copy(elb, dacc.at[dstv.at[j]], add=True)
        return carry

    lax.fori_loop(0, NCH, step, 0)
    plsc.subcore_barrier()
    _dump_stripe(dacc, den_out, c, s)


def _sc_edge(elq, erq, eep, idxs, zeros128):
    fn = pl.kernel(
        _edge_body,
        out_type=(jax.ShapeDtypeStruct((EPAD // 8, 128), _f32),
                  jax.ShapeDtypeStruct((2, NP, 128), _f32)),
        mesh=_mesh,
        scratch_types=[
            pltpu.VMEM((NCH, CH), jnp.int32),
            pltpu.VMEM((NCH, CH), jnp.int32),
            pltpu.VMEM((CH, 128), _f32),
            pltpu.VMEM((CH, 128), _f32),
            pltpu.VMEM((CH // 8, 128), _f32),
            pltpu.VMEM((CH // 8, 128), _f32),
            pltpu.VMEM_SHARED((NP, 128), _f32),
            pltpu.SemaphoreType.DMA,
            pltpu.SemaphoreType.DMA,
        ],
    )
    return fn(elq, erq, eep, idxs, zeros128)


def _gatagg_body(featf, w_hbm, idxs, zeros128, out,
                 srcv, dstv, sidx0, sidx1, gb0, gb1, wb, acc, sem0, sem1,
                 ssem0, ssem1):
    c = lax.axis_index("c")
    s = lax.axis_index("s")
    w = _wid(c, s)
    pltpu.sync_copy(idxs.at[w, 0], srcv)
    pltpu.sync_copy(idxs.at[w, 1], dstv)

    def shift_into(dst_ref, j, h):
        def shift(k, carry2):
            dst_ref[pl.ds(k * 16, 16)] = srcv[j, pl.ds(k * 16, 16)] + h * NP
            return carry2

        lax.fori_loop(0, CH // 16, shift, 0)

    def scale_scatter(gb, sidx, j, h, ssem):
        wrowi = w * (EPT // 8) + j * (CH // 8)
        pltpu.make_async_copy(featf.at[sidx], gb, sem0 if gb is gb0 else sem1
                              ).wait()
        pltpu.sync_copy(w_hbm.at[pl.ds(wrowi, CH // 8)], wb)

        def inner(r, carry2):
            for e8 in range(8):
                wrow = wb[r, pl.ds(e8 * 16, 16)]
                hidx = lax.iota(jnp.int32, 16) * 0 + h
                wv = lax.gather(
                    wrow, hidx[:, None],
                    lax.GatherDimensionNumbers(
                        offset_dims=(), collapsed_slice_dims=(0,),
                        start_index_map=(0,)),
                    (1,), mode=lax.GatherScatterMode.PROMISE_IN_BOUNDS)
                e = r * 8 + e8
                for k in range(8):
                    gb[e, pl.ds(k * 16, 16)] = gb[e, pl.ds(k * 16, 16)] * wv
            return carry2

        lax.fori_loop(0, CH // 8, inner, 0)
        pltpu.async_copy(gb, acc.at[dstv.at[j]], ssem, add=True)

    for h in range(8):
        _zero_stripe(zeros128, acc, s)
        plsc.subcore_barrier()

        shift_into(sidx0, 0, h)
        pltpu.async_copy(featf.at[sidx0], gb0, sem0)

        def step(j2, carry):
            a = 2 * j2
            b = a + 1
            shift_into(sidx1, b, h)
            pltpu.async_copy(featf.at[sidx1], gb1, sem1)
            scale_scatter(gb0, sidx0, a, h, ssem0)
            pltpu.make_async_copy(gb0, acc.at[dstv.at[a]], ssem0).wait()

            @pl.when(j2 < NCH // 2 - 1)
            def _():
                shift_into(sidx0, a + 2, h)
                pltpu.async_copy(featf.at[sidx0], gb0, sem0)

            scale_scatter(gb1, sidx1, b, h, ssem1)
            pltpu.make_async_copy(gb1, acc.at[dstv.at[b]], ssem1).wait()
            return carry

        lax.fori_loop(0, NCH // 2, step, 0)
        plsc.subcore_barrier()
        _dump_stripe(acc, out.at[:, h], c, s)


def _sc_gatagg(featf, w_hbm, idxs, zeros128):
    fn = pl.kernel(
        _gatagg_body,
        out_type=jax.ShapeDtypeStruct((2, 8, NP, 128), _f32),
        mesh=_mesh,
        scratch_types=[
            pltpu.VMEM((NCH, CH), jnp.int32),
            pltpu.VMEM((NCH, CH), jnp.int32),
            pltpu.VMEM((CH,), jnp.int32),
            pltpu.VMEM((CH,), jnp.int32),
            pltpu.VMEM((CH, 128), _f32),
            pltpu.VMEM((CH, 128), _f32),
            pltpu.VMEM((CH // 8, 128), _f32),
            pltpu.VMEM_SHARED((NP, 128), _f32),
            pltpu.SemaphoreType.DMA,
            pltpu.SemaphoreType.DMA,
            pltpu.SemaphoreType.DMA,
            pltpu.SemaphoreType.DMA,
        ],
    )
    return fn(featf, w_hbm, idxs, zeros128)


def _rows_mask(i):
    r = lax.broadcasted_iota(jnp.int32, (RB, 1), 0) + i * RB
    return r < N


def _bn_apply(y, stats, g, be):
    mu = stats[0:1, :] / N
    var = stats[1:2, :] / N - mu * mu
    return (y - mu) * lax.rsqrt(var + 1e-5) * g + be


def _acc_stats(i, y, acc, stats_ref):
    @pl.when(i == 0)
    def _():
        acc[...] = jnp.zeros_like(acc)

    acc[0:1, :] += jnp.sum(y, axis=0, keepdims=True)
    acc[1:2, :] += jnp.sum(y * y, axis=0, keepdims=True)

    @pl.when(i == GN - 1)
    def _():
        stats_ref[...] = acc[...]


def _bspec(shape, imap):
    return pl.BlockSpec(shape, imap)


def _t1_body(deg, xp, W1, t1, no):
    i = pl.program_id(0)
    m = _rows_mask(i)
    od = deg[0, :, 0:1] + deg[1, :, 0:1] + 1.0
    idg = deg[0, :, 1:2] + deg[1, :, 1:2] + 1.0
    on = jnp.where(m, lax.rsqrt(jnp.maximum(od, 1.0)), 0.0)
    inn = jnp.where(m, lax.rsqrt(jnp.maximum(idg, 1.0)), 0.0)
    no[...] = jnp.concatenate([on, inn, jnp.zeros((RB, 6), _f32)], axis=1)
    t = (xp[...] * on) @ W1[...]
    t1[...] = jnp.concatenate([t, jnp.zeros((RB, 64), _f32)], axis=1)


def _tc_t1(deg, xp, W1):
    return pl.pallas_call(
        _t1_body,
        grid=(GN,),
        in_specs=[
            _bspec((2, RB, 128), lambda i: (0, i, 0)),
            _bspec((RB, 128), lambda i: (i, 0)),
            _bspec((128, 64), lambda i: (0, 0)),
        ],
        out_specs=[
            _bspec((RB, 128), lambda i: (i, 0)),
            _bspec((RB, 8), lambda i: (i, 0)),
        ],
        out_shape=[jax.ShapeDtypeStruct((NP, 128), _f32),
                   jax.ShapeDtypeStruct((NP, 8), _f32)],
    )(deg, xp, W1)


def _post1_body(s1p, t1, no, b1, y1, stats, acc):
    i = pl.program_id(0)
    m = _rows_mask(i)
    inn = no[:, 1:2]
    y = (s1p[0, :, :64] + s1p[1, :, :64] + t1[:, :64]) * inn + b1[...]
    y = jnp.where(m & (y > 0), y, 0.0)
    y1[...] = y
    _acc_stats(i, y, acc, stats)


def _tc_post1(s1p, t1, no, b1):
    return pl.pallas_call(
        _post1_body,
        grid=(GN,),
        in_specs=[
            _bspec((2, RB, 128), lambda i: (0, i, 0)),
            _bspec((RB, 128), lambda i: (i, 0)),
            _bspec((RB, 8), lambda i: (i, 0)),
            _bspec((1, 64), lambda i: (0, 0)),
        ],
        out_specs=[
            _bspec((RB, 64), lambda i: (i, 0)),
            _bspec((2, 64), lambda i: (0, 0)),
        ],
        out_shape=[jax.ShapeDtypeStruct((NP, 64), _f32),
                   jax.ShapeDtypeStruct((2, 64), _f32)],
        scratch_shapes=[pltpu.VMEM((2, 64), _f32)],
    )(s1p, t1, no, b1)


def _u2_body(y1, stats, g1, be1, no, u2):
    i = pl.program_id(0)
    m = _rows_mask(i)
    on = no[:, 0:1]
    u = _bn_apply(y1[...], stats[...], g1[...], be1[...]) * on
    u = jnp.where(m, u, 0.0)
    u2[...] = jnp.concatenate([u, jnp.zeros((RB, 64), _f32)], axis=1)


def _tc_u2(y1, stats, g1, be1, no):
    return pl.pallas_call(
        _u2_body,
        grid=(GN,),
        in_specs=[
            _bspec((RB, 64), lambda i: (i, 0)),
            _bspec((2, 64), lambda i: (0, 0)),
            _bspec((1, 64), lambda i: (0, 0)),
            _bspec((1, 64), lambda i: (0, 0)),
            _bspec((RB, 8), lambda i: (i, 0)),
        ],
        out_specs=[_bspec((RB, 128), lambda i: (i, 0))],
        out_shape=[jax.ShapeDtypeStruct((NP, 128), _f32)],
    )(y1, stats, g1, be1, no)[0]


def _mid2_body(s2p, u2, no, W2, b2, y2, stats, acc):
    i = pl.program_id(0)
    m = _rows_mask(i)
    inn = no[:, 1:2]
    s2 = s2p[0, :, :64] + s2p[1, :, :64] + u2[:, :64]
    y = (s2 * inn) @ W2[...] + b2[...]
    y = jnp.where(m & (y > 0), y, 0.0)
    y2[...] = y
    _acc_stats(i, y, acc, stats)


def _tc_mid2(s2p, u2, no, W2, b2):
    return pl.pallas_call(
        _mid2_body,
        grid=(GN,),
        in_specs=[
            _bspec((2, RB, 128), lambda i: (0, i, 0)),
            _bspec((RB, 128), lambda i: (i, 0)),
            _bspec((RB, 8), lambda i: (i, 0)),
            _bspec((64, 256), lambda i: (0, 0)),
            _bspec((1, 256), lambda i: (0, 0)),
        ],
        out_specs=[
            _bspec((RB, 256), lambda i: (i, 0)),
            _bspec((2, 256), lambda i: (0, 0)),
        ],
        out_shape=[jax.ShapeDtypeStruct((NP, 256), _f32),
                   jax.ShapeDtypeStruct((2, 256), _f32)],
        scratch_shapes=[pltpu.VMEM((2, 256), _f32)],
    )(s2p, u2, no, W2, b2)


def _u3_body(y2, stats, g2, be2, no, u3t):
    i = pl.program_id(0)
    m = _rows_mask(i)
    on = no[:, 0:1]
    u = _bn_apply(y2[...], stats[...], g2[...], be2[...]) * on
    u = jnp.where(m, u, 0.0)
    u3t[...] = jnp.stack([u[:, :128], u[:, 128:]], axis=0)


def _tc_u3(y2, stats, g2, be2, no):
    return pl.pallas_call(
        _u3_body,
        grid=(GN,),
        in_specs=[
            _bspec((RB, 256), lambda i: (i, 0)),
            _bspec((2, 256), lambda i: (0, 0)),
            _bspec((1, 256), lambda i: (0, 0)),
            _bspec((1, 256), lambda i: (0, 0)),
            _bspec((RB, 8), lambda i: (i, 0)),
        ],
        out_specs=[_bspec((2, RB, 128), lambda i: (0, i, 0))],
        out_shape=[jax.ShapeDtypeStruct((2, NP, 128), _f32)],
    )(y2, stats, g2, be2, no)[0]


def _mid3_body(s3pa, s3pb, u3t, no, W3, b3, y3, stats, acc):
    i = pl.program_id(0)
    m = _rows_mask(i)
    inn = no[:, 1:2]
    sa = s3pa[0] + s3pa[1] + u3t[0]
    sb = s3pb[0] + s3pb[1] + u3t[1]
    s3 = jnp.concatenate([sa, sb], axis=1)
    y = (s3 * inn) @ W3[...] + b3[...]
    y = jnp.where(m & (y > 0), y, 0.0)
    y3[...] = y
    _acc_stats(i, y, acc, stats)


def _tc_mid3(s3pa, s3pb, u3t, no, W3, b3):
    return pl.pallas_call(
        _mid3_body,
        grid=(GN,),
        in_specs=[
            _bspec((2, RB, 128), lambda i: (0, i, 0)),
            _bspec((2, RB, 128), lambda i: (0, i, 0)),
            _bspec((2, RB, 128), lambda i: (0, i, 0)),
            _bspec((RB, 8), lambda i: (i, 0)),
            _bspec((256, 512), lambda i: (0, 0)),
            _bspec((1, 512), lambda i: (0, 0)),
        ],
        out_specs=[
            _bspec((RB, 512), lambda i: (i, 0)),
            _bspec((2, 512), lambda i: (0, 0)),
        ],
        out_shape=[jax.ShapeDtypeStruct((NP, 512), _f32),
                   jax.ShapeDtypeStruct((2, 512), _f32)],
        scratch_shapes=[pltpu.VMEM((2, 512), _f32)],
    )(s3pa, s3pb, u3t, no, W3, b3)


def _featres_body(y3, stats, g3, be3, Wgh, Wrh, featf, resf):
    i = pl.program_id(0)
    m = _rows_mask(i)
    h3 = _bn_apply(y3[...], stats[...], g3[...], be3[...])
    h3 = jnp.where(m, h3, 0.0)
    featf[0] = h3 @ Wgh[...]
    resf[0] = h3 @ Wrh[...]


def _tc_featres(y3, stats, g3, be3, Wg, Wres):
    return pl.pallas_call(
        _featres_body,
        grid=(GN, 8),
        in_specs=[
            _bspec((RB, 512), lambda i, h: (i, 0)),
            _bspec((2, 512), lambda i, h: (0, 0)),
            _bspec((1, 512), lambda i, h: (0, 0)),
            _bspec((1, 512), lambda i, h: (0, 0)),
            _bspec((512, 128), lambda i, h: (0, h)),
            _bspec((512, 128), lambda i, h: (0, h)),
        ],
        out_specs=[
            _bspec((1, RB, 128), lambda i, h: (h, i, 0)),
            _bspec((1, RB, 128), lambda i, h: (h, i, 0)),
        ],
        out_shape=[jax.ShapeDtypeStruct((8, NP, 128), _f32),
                   jax.ShapeDtypeStruct((8, NP, 128), _f32)],
    )(y3, stats, g3, be3, Wg, Wres)


def _eler_body(y3, stats, g3, be3, Wg, al, ar, elq, erq, wl):
    i = pl.program_id(0)
    m = _rows_mask(i)
    h3 = _bn_apply(y3[...], stats[...], g3[...], be3[...])
    h3 = jnp.where(m, h3, 0.0)
    Wg3 = Wg[...].reshape(512, 8, 128)
    Al = jnp.sum(Wg3 * al[...][None, :, :], axis=-1)
    Ar = jnp.sum(Wg3 * ar[...][None, :, :], axis=-1)
    el = h3 @ Al
    er = h3 @ Ar
    z120 = jnp.zeros((RB, 120), _f32)
    elq[...] = jnp.concatenate([el, z120], axis=1)
    erq[...] = jnp.concatenate([er, z120], axis=1)
    v = el + er
    v = jnp.where(v > 0, v, 0.2 * v)
    wl[...] = jnp.exp(v)


def _tc_eler(y3, stats, g3, be3, Wg, al, ar):
    return pl.pallas_call(
        _eler_body,
        grid=(GN,),
        in_specs=[
            _bspec((RB, 512), lambda i: (i, 0)),
            _bspec((2, 512), lambda i: (0, 0)),
            _bspec((1, 512), lambda i: (0, 0)),
            _bspec((1, 512), lambda i: (0, 0)),
            _bspec((512, 1024), lambda i: (0, 0)),
            _bspec((8, 128), lambda i: (0, 0)),
            _bspec((8, 128), lambda i: (0, 0)),
        ],
        out_specs=[
            _bspec((RB, 128), lambda i: (i, 0)),
            _bspec((RB, 128), lambda i: (i, 0)),
            _bspec((RB, 8), lambda i: (i, 0)),
        ],
        out_shape=[jax.ShapeDtypeStruct((NP, 128), _f32),
                   jax.ShapeDtypeStruct((NP, 128), _f32),
                   jax.ShapeDtypeStruct((NP, 8), _f32)],
    )(y3, stats, g3, be3, Wg, al, ar)


def _ee_body2(efq, We, ae, eep):
    Ae = jnp.sum(We[...].reshape(2, 8, 128) * ae[...][None, :, :], axis=-1)
    Ae16 = jnp.concatenate([Ae, jnp.zeros((2, 8), _f32)], axis=1)
    rows = []
    for a in range(8):
        parts = []
        if a > 0:
            parts.append(jnp.zeros((2, 16 * a), _f32))
        parts.append(Ae16)
        if a < 7:
            parts.append(jnp.zeros((2, 112 - 16 * a), _f32))
        rows.append(jnp.concatenate(parts, axis=1))
    B = jnp.concatenate(rows, axis=0)
    eep[...] = efq[...] @ B


def _tc_ee(efq, We, ae):
    return pl.pallas_call(
        _ee_body2,
        grid=(8,),
        in_specs=[
            _bspec((EB // 8, 16), lambda i: (i, 0)),
            _bspec((2, 1024), lambda i: (0, 0)),
            _bspec((8, 128), lambda i: (0, 0)),
        ],
        out_specs=[_bspec((EB // 8, 128), lambda i: (i, 0))],
        out_shape=[jax.ShapeDtypeStruct((EPAD // 8, 128), _f32)],
    )(efq, We, ae)[0]


def _final1_body(hofs, rstph, featfh, resfh, denp, wl, bgrh, zp, stats, acc):
    i = pl.program_id(0)
    m = _rows_mask(i)
    cols = []
    for hh in range(4):
        hg = hofs + hh
        wlh = wl[:, hg:hg + 1]
        den = denp[0, :, hg:hg + 1] + denp[1, :, hg:hg + 1] + wlh
        r = (rstph[0, hh] + rstph[1, hh] + featfh[hh] * wlh) / den
        r = r + resfh[hh] + bgrh[hh:hh + 1, :]
        cols.append(r)
    z = jnp.concatenate(cols, axis=1)
    z = jnp.where(m, z, 0.0)
    zp[...] = z
    _acc_stats(i, z, acc, stats)


def _tc_final1h(hofs, rstph, featfh, resfh, denp, wl, bgrh):
    return pl.pallas_call(
        functools.partial(_final1_body, hofs),
        grid=(GN,),
        in_specs=[
            _bspec((2, 4, RB, 128), lambda i: (0, 0, i, 0)),
            _bspec((4, RB, 128), lambda i: (0, i, 0)),
            _bspec((4, RB, 128), lambda i: (0, i, 0)),
            _bspec((2, RB, 128), lambda i: (0, i, 0)),
            _bspec((RB, 8), lambda i: (i, 0)),
            _bspec((4, 128), lambda i: (0, 0)),
        ],
        out_specs=[
            _bspec((RB, 512), lambda i: (i, 0)),
            _bspec((2, 512), lambda i: (0, 0)),
        ],
        out_shape=[jax.ShapeDtypeStruct((NP, 512), _f32),
                   jax.ShapeDtypeStruct((2, 512), _f32)],
        scratch_shapes=[pltpu.VMEM((2, 512), _f32)],
    )(rstph, featfh, resfh, denp, wl, bgrh)


def _final2_body(zpa, zpb, sta, stb, g5, be5, Wa1, ba1, Wa2t, ba2,
                 Wc1, bc1, Wc2t, bc2, a_out, c_out):
    zp = jnp.concatenate([zpa[...], zpb[...]], axis=1)
    stats = jnp.concatenate([sta[...], stb[...]], axis=1)
    z2 = _bn_apply(zp, stats, g5[...], be5[...])
    a1 = jnp.maximum(z2 @ Wa1[...] + ba1[...], 0.0)
    a_out[...] = jnp.sum(a1 * Wa2t[...], axis=1, keepdims=True) + ba2[...]
    c1 = jnp.maximum(z2 @ Wc1[...] + bc1[...], 0.0)
    c_out[...] = jnp.sum(c1 * Wc2t[...], axis=1, keepdims=True) + bc2[...]


def _tc_final2(zpa, zpb, sta, stb, g5, be5, Wa1, ba1, Wa2t, ba2,
               Wc1, bc1, Wc2t, bc2):
    return pl.pallas_call(
        _final2_body,
        grid=(GN,),
        in_specs=[
            _bspec((RB, 512), lambda i: (i, 0)),
            _bspec((RB, 512), lambda i: (i, 0)),
            _bspec((2, 512), lambda i: (0, 0)),
            _bspec((2, 512), lambda i: (0, 0)),
            _bspec((1, 1024), lambda i: (0, 0)),
            _bspec((1, 1024), lambda i: (0, 0)),
            _bspec((1024, 128), lambda i: (0, 0)),
            _bspec((1, 128), lambda i: (0, 0)),
            _bspec((1, 128), lambda i: (0, 0)),
            _bspec((1, 1), lambda i: (0, 0)),
            _bspec((1024, 128), lambda i: (0, 0)),
            _bspec((1, 128), lambda i: (0, 0)),
            _bspec((1, 128), lambda i: (0, 0)),
            _bspec((1, 1), lambda i: (0, 0)),
        ],
        out_specs=[
            _bspec((RB, 1), lambda i: (i, 0)),
            _bspec((RB, 1), lambda i: (i, 0)),
        ],
        out_shape=[jax.ShapeDtypeStruct((N, 1), _f32),
                   jax.ShapeDtypeStruct((N, 1), _f32)],
    )(zpa, zpb, sta, stb, g5, be5, Wa1, ba1, Wa2t, ba2, Wc1, bc1, Wc2t, bc2)


def kernel(x, edge_index, edge_feat, W1, b1, W2, b2, W3, b3,
           g1, be1, g2, be2, g3, be3, g5, be5,
           Wg, We, al, ar, ae, Wres, bg,
           Wa1, ba1, Wa2, ba2, Wc1, bc1, Wc2, bc2):
    i32 = jnp.int32
    r1 = lambda v: v.reshape(1, -1)
    src = edge_index[0]
    dst = edge_index[1]
    padn = EPAD - E
    srcp = jnp.concatenate([src, jnp.full((padn,), DUMMY, i32)])
    dstp = jnp.concatenate([dst, jnp.full((padn,), DUMMY, i32)])
    idxs = jnp.stack([srcp.reshape(NW, NCH, CH),
                      dstp.reshape(NW, NCH, CH)], axis=1)

    ev0 = jnp.zeros((CH, 128), _f32).at[:, 0].set(1.0)
    ev1 = jnp.zeros((CH, 128), _f32).at[:, 1].set(1.0)
    zeros16 = jnp.zeros((NP, 16), _f32)
    zeros128 = jnp.zeros((NP, 128), _f32)
    xp = jnp.pad(x, ((0, NP - N), (0, 0)))
    efq = jnp.pad(edge_feat, ((0, padn), (0, 0))).reshape(EPAD // 8, 16)

    deg = _sc_degrees(idxs, ev0, ev1, zeros128)
    t1, no = _tc_t1(deg, xp, W1)

    s1p = _sc_agg(t1, idxs, zeros128)
    y1, st1 = _tc_post1(s1p, t1, no, r1(b1))
    u2 = _tc_u2(y1, st1, r1(g1), r1(be1), no)

    s2p = _sc_agg(u2, idxs, zeros128)
    y2, st2 = _tc_mid2(s2p, u2, no, W2, r1(b2))
    u3t = _tc_u3(y2, st2, r1(g2), r1(be2), no)

    s3pa = _sc_agg(u3t[0], idxs, zeros128)
    s3pb = _sc_agg(u3t[1], idxs, zeros128)
    y3, st3 = _tc_mid3(s3pa, s3pb, u3t, no, W3, r1(b3))

    featf, resf = _tc_featres(y3, st3, r1(g3), r1(be3), Wg, Wres)
    elq, erq, wl = _tc_eler(y3, st3, r1(g3), r1(be3), Wg, al, ar)
    eep = _tc_ee(efq, We, ae)

    w_hbm, denp = _sc_edge(elq, erq, eep, idxs, zeros128)
    rstp = _sc_gatagg(featf.reshape(8 * NP, 128), w_hbm, idxs, zeros128)

    bgr = bg.reshape(8, 128)
    zpa, sta = _tc_final1h(0, rstp[:, 0:4], featf[0:4], resf[0:4], denp, wl,
                           bgr[0:4])
    zpb, stb = _tc_final1h(4, rstp[:, 4:8], featf[4:8], resf[4:8], denp, wl,
                           bgr[4:8])
    a, c = _tc_final2(zpa, zpb, sta, stb, r1(g5), r1(be5), Wa1, r1(ba1),
                      r1(Wa2[:, 0]), ba2.reshape(1, 1), Wc1, r1(bc1),
                      r1(Wc2[:, 0]), bc2.reshape(1, 1))
    return (a, c)


# speedup vs baseline: 1.0556x; 1.0556x over previous
import functools

import jax
import jax.numpy as jnp
from jax import lax
from jax.experimental import pallas as pl
from jax.experimental.pallas import tpu as pltpu
from jax.experimental.pallas import tpu_sc as plsc

N = 10000
E = 160000
NW = 32
EPT = 5120
CH = 128
NCH = EPT // CH
EPAD = NW * EPT
NP = 10112
RPT = NP // 16
DUMMY = N
RB = 1264
GN = 8
EB = EPAD // 8

_f32 = jnp.float32
_mesh = plsc.VectorSubcoreMesh(core_axis_name="c", subcore_axis_name="s")


def _wid(c, s):
    return s * 2 + c


def _zero_stripe(zeros_hbm, accum, s):
    pltpu.sync_copy(zeros_hbm.at[pl.ds(s * RPT, RPT)],
                    accum.at[pl.ds(s * RPT, RPT)])


def _dump_stripe(accum, out_hbm, c, s):
    pltpu.sync_copy(accum.at[pl.ds(s * RPT, RPT)],
                    out_hbm.at[c, pl.ds(s * RPT, RPT)])


def _deg_body(idxs, ev0, ev1, zeros128, out, srcv, dstv, vb0, vb1, acc):
    c = lax.axis_index("c")
    s = lax.axis_index("s")
    w = _wid(c, s)
    _zero_stripe(zeros128, acc, s)
    pltpu.sync_copy(ev0, vb0)
    pltpu.sync_copy(ev1, vb1)
    pltpu.sync_copy(idxs.at[w, 0], srcv)
    pltpu.sync_copy(idxs.at[w, 1], dstv)
    plsc.subcore_barrier()

    def step(j, carry):
        pltpu.sync_copy(vb0, acc.at[srcv.at[j]], add=True)
        pltpu.sync_copy(vb1, acc.at[dstv.at[j]], add=True)
        return carry

    lax.fori_loop(0, NCH, step, 0)
    plsc.subcore_barrier()
    _dump_stripe(acc, out, c, s)


def _sc_degrees(idxs, ev0, ev1, zeros128):
    fn = pl.kernel(
        _deg_body,
        out_type=jax.ShapeDtypeStruct((2, NP, 128), _f32),
        mesh=_mesh,
        scratch_types=[
            pltpu.VMEM((NCH, CH), jnp.int32),
            pltpu.VMEM((NCH, CH), jnp.int32),
            pltpu.VMEM((CH, 128), _f32),
            pltpu.VMEM((CH, 128), _f32),
            pltpu.VMEM_SHARED((NP, 128), _f32),
        ],
    )
    return fn(idxs, ev0, ev1, zeros128)


def _agg_body(table, idxs, zeros128, out, srcv, dstv, gb0, gb1, accum,
              sem0, sem1):
    c = lax.axis_index("c")
    s = lax.axis_index("s")
    w = _wid(c, s)
    _zero_stripe(zeros128, accum, s)
    pltpu.sync_copy(idxs.at[w, 0], srcv)
    pltpu.sync_copy(idxs.at[w, 1], dstv)
    plsc.subcore_barrier()

    pltpu.async_copy(table.at[srcv.at[0]], gb0, sem0)

    def step(j2, carry):
        a = 2 * j2
        b = a + 1
        pltpu.async_copy(table.at[srcv.at[b]], gb1, sem1)
        pltpu.make_async_copy(table.at[srcv.at[a]], gb0, sem0).wait()
        pltpu.sync_copy(gb0, accum.at[dstv.at[a]], add=True)

        @pl.when(j2 < NCH // 2 - 1)
        def _():
            pltpu.async_copy(table.at[srcv.at[a + 2]], gb0, sem0)

        pltpu.make_async_copy(table.at[srcv.at[b]], gb1, sem1).wait()
        pltpu.sync_copy(gb1, accum.at[dstv.at[b]], add=True)
        return carry

    lax.fori_loop(0, NCH // 2, step, 0)
    plsc.subcore_barrier()
    _dump_stripe(accum, out, c, s)


def _sc_agg(table, idxs, zeros128):
    fn = pl.kernel(
        _agg_body,
        out_type=jax.ShapeDtypeStruct((2, NP, 128), _f32),
        mesh=_mesh,
        scratch_types=[
            pltpu.VMEM((NCH, CH), jnp.int32),
            pltpu.VMEM((NCH, CH), jnp.int32),
            pltpu.VMEM((CH, 128), _f32),
            pltpu.VMEM((CH, 128), _f32),
            pltpu.VMEM_SHARED((NP, 128), _f32),
            pltpu.SemaphoreType.DMA,
            pltpu.SemaphoreType.DMA,
        ],
    )
    return fn(table, idxs, zeros128)


def _edge_body(elq, erq, eep, idxs, zeros128, w_out, den_out,
               srcv, dstv, elb, erb, eeb, wb, dacc, sem, sem2):
    c = lax.axis_index("c")
    s = lax.axis_index("s")
    w = _wid(c, s)
    _zero_stripe(zeros128, dacc, s)
    pltpu.sync_copy(idxs.at[w, 0], srcv)
    pltpu.sync_copy(idxs.at[w, 1], dstv)
    plsc.subcore_barrier()

    def step(j, carry):
        wrow = w * (EPT // 8) + j * (CH // 8)
        cp0 = pltpu.async_copy(elq.at[srcv.at[j]], elb, sem)
        cp1 = pltpu.async_copy(erq.at[dstv.at[j]], erb, sem2)
        pltpu.sync_copy(eep.at[pl.ds(wrow, CH // 8)], eeb)
        cp0.wait()
        cp1.wait()

        def inner(r, carry2):
            for e8 in range(8):
                e = r * 8 + e8
                v = (elb[e, pl.ds(0, 16)] + erb[e, pl.ds(0, 16)]
                     + eeb[r, pl.ds(e8 * 16, 16)])
                v = jnp.where(v > 0, v, 0.2 * v)
                v = jnp.exp(v)
                wb[r, pl.ds(e8 * 16, 16)] = v
                elb[e, pl.ds(0, 16)] = v
            return carry2

        lax.fori_loop(0, CH // 8, inner, 0)
        pltpu.sync_copy(wb, w_out.at[pl.ds(wrow, CH // 8)])
        pltpu.sync_copy(elb, dacc.at[dstv.at[j]], add=True)
        return carry

    lax.fori_loop(0, NCH, step, 0)
    plsc.subcore_barrier()
    _dump_stripe(dacc, den_out, c, s)


def _sc_edge(elq, erq, eep, idxs, zeros128):
    fn = pl.kernel(
        _edge_body,
        out_type=(jax.ShapeDtypeStruct((EPAD // 8, 128), _f32),
                  jax.ShapeDtypeStruct((2, NP, 128), _f32)),
        mesh=_mesh,
        scratch_types=[
            pltpu.VMEM((NCH, CH), jnp.int32),
            pltpu.VMEM((NCH, CH), jnp.int32),
            pltpu.VMEM((CH, 128), _f32),
            pltpu.VMEM((CH, 128), _f32),
            pltpu.VMEM((CH // 8, 128), _f32),
            pltpu.VMEM((CH // 8, 128), _f32),
            pltpu.VMEM_SHARED((NP, 128), _f32),
            pltpu.SemaphoreType.DMA,
            pltpu.SemaphoreType.DMA,
        ],
    )
    return fn(elq, erq, eep, idxs, zeros128)


def _gatagg_body(featf, w_hbm, idxs, zeros128, out,
                 srcv, dstv, sidx0, sidx1, gb0, gb1, wb, acc, sem0, sem1):
    c = lax.axis_index("c")
    s = lax.axis_index("s")
    w = _wid(c, s)
    pltpu.sync_copy(idxs.at[w, 0], srcv)
    pltpu.sync_copy(idxs.at[w, 1], dstv)

    def shift_into(dst_ref, j, h):
        def shift(k, carry2):
            dst_ref[pl.ds(k * 16, 16)] = srcv[j, pl.ds(k * 16, 16)] + h * NP
            return carry2

        lax.fori_loop(0, CH // 16, shift, 0)

    def scale_scatter(gb, sidx, j, h):
        wrowi = w * (EPT // 8) + j * (CH // 8)
        pltpu.make_async_copy(featf.at[sidx], gb, sem0 if gb is gb0 else sem1
                              ).wait()
        pltpu.sync_copy(w_hbm.at[pl.ds(wrowi, CH // 8)], wb)

        def inner(r, carry2):
            for e8 in range(8):
                wrow = wb[r, pl.ds(e8 * 16, 16)]
                hidx = lax.iota(jnp.int32, 16) * 0 + h
                wv = lax.gather(
                    wrow, hidx[:, None],
                    lax.GatherDimensionNumbers(
                        offset_dims=(), collapsed_slice_dims=(0,),
                        start_index_map=(0,)),
                    (1,), mode=lax.GatherScatterMode.PROMISE_IN_BOUNDS)
                e = r * 8 + e8
                for k in range(8):
                    gb[e, pl.ds(k * 16, 16)] = gb[e, pl.ds(k * 16, 16)] * wv
            return carry2

        lax.fori_loop(0, CH // 8, inner, 0)
        pltpu.sync_copy(gb, acc.at[dstv.at[j]], add=True)

    for h in range(8):
        _zero_stripe(zeros128, acc, s)
        plsc.subcore_barrier()

        shift_into(sidx0, 0, h)
        pltpu.async_copy(featf.at[sidx0], gb0, sem0)

        def step(j2, carry):
            a = 2 * j2
            b = a + 1
            shift_into(sidx1, b, h)
            pltpu.async_copy(featf.at[sidx1], gb1, sem1)
            scale_scatter(gb0, sidx0, a, h)

            @pl.when(j2 < NCH // 2 - 1)
            def _():
                shift_into(sidx0, a + 2, h)
                pltpu.async_copy(featf.at[sidx0], gb0, sem0)

            scale_scatter(gb1, sidx1, b, h)
            return carry

        lax.fori_loop(0, NCH // 2, step, 0)
        plsc.subcore_barrier()
        _dump_stripe(acc, out.at[:, h], c, s)


def _sc_gatagg(featf, w_hbm, idxs, zeros128):
    fn = pl.kernel(
        _gatagg_body,
        out_type=jax.ShapeDtypeStruct((2, 8, NP, 128), _f32),
        mesh=_mesh,
        scratch_types=[
            pltpu.VMEM((NCH, CH), jnp.int32),
            pltpu.VMEM((NCH, CH), jnp.int32),
            pltpu.VMEM((CH,), jnp.int32),
            pltpu.VMEM((CH,), jnp.int32),
            pltpu.VMEM((CH, 128), _f32),
            pltpu.VMEM((CH, 128), _f32),
            pltpu.VMEM((CH // 8, 128), _f32),
            pltpu.VMEM_SHARED((NP, 128), _f32),
            pltpu.SemaphoreType.DMA,
            pltpu.SemaphoreType.DMA,
        ],
    )
    return fn(featf, w_hbm, idxs, zeros128)


def _rows_mask(i):
    r = lax.broadcasted_iota(jnp.int32, (RB, 1), 0) + i * RB
    return r < N


def _bn_apply(y, stats, g, be):
    mu = stats[0:1, :] / N
    var = stats[1:2, :] / N - mu * mu
    return (y - mu) * lax.rsqrt(var + 1e-5) * g + be


def _acc_stats(i, y, acc, stats_ref):
    @pl.when(i == 0)
    def _():
        acc[...] = jnp.zeros_like(acc)

    acc[0:1, :] += jnp.sum(y, axis=0, keepdims=True)
    acc[1:2, :] += jnp.sum(y * y, axis=0, keepdims=True)

    @pl.when(i == GN - 1)
    def _():
        stats_ref[...] = acc[...]


def _bspec(shape, imap):
    return pl.BlockSpec(shape, imap)


def _t1_body(deg, xp, W1, t1, no):
    i = pl.program_id(0)
    m = _rows_mask(i)
    od = deg[0, :, 0:1] + deg[1, :, 0:1] + 1.0
    idg = deg[0, :, 1:2] + deg[1, :, 1:2] + 1.0
    on = jnp.where(m, lax.rsqrt(jnp.maximum(od, 1.0)), 0.0)
    inn = jnp.where(m, lax.rsqrt(jnp.maximum(idg, 1.0)), 0.0)
    no[...] = jnp.concatenate([on, inn, jnp.zeros((RB, 6), _f32)], axis=1)
    t = (xp[...] * on) @ W1[...]
    t1[...] = jnp.concatenate([t, jnp.zeros((RB, 64), _f32)], axis=1)


def _tc_t1(deg, xp, W1):
    return pl.pallas_call(
        _t1_body,
        grid=(GN,),
        in_specs=[
            _bspec((2, RB, 128), lambda i: (0, i, 0)),
            _bspec((RB, 128), lambda i: (i, 0)),
            _bspec((128, 64), lambda i: (0, 0)),
        ],
        out_specs=[
            _bspec((RB, 128), lambda i: (i, 0)),
            _bspec((RB, 8), lambda i: (i, 0)),
        ],
        out_shape=[jax.ShapeDtypeStruct((NP, 128), _f32),
                   jax.ShapeDtypeStruct((NP, 8), _f32)],
    )(deg, xp, W1)


def _post1_body(s1p, t1, no, b1, y1, stats, acc):
    i = pl.program_id(0)
    m = _rows_mask(i)
    inn = no[:, 1:2]
    y = (s1p[0, :, :64] + s1p[1, :, :64] + t1[:, :64]) * inn + b1[...]
    y = jnp.where(m & (y > 0), y, 0.0)
    y1[...] = y
    _acc_stats(i, y, acc, stats)


def _tc_post1(s1p, t1, no, b1):
    return pl.pallas_call(
        _post1_body,
        grid=(GN,),
        in_specs=[
            _bspec((2, RB, 128), lambda i: (0, i, 0)),
            _bspec((RB, 128), lambda i: (i, 0)),
            _bspec((RB, 8), lambda i: (i, 0)),
            _bspec((1, 64), lambda i: (0, 0)),
        ],
        out_specs=[
            _bspec((RB, 64), lambda i: (i, 0)),
            _bspec((2, 64), lambda i: (0, 0)),
        ],
        out_shape=[jax.ShapeDtypeStruct((NP, 64), _f32),
                   jax.ShapeDtypeStruct((2, 64), _f32)],
        scratch_shapes=[pltpu.VMEM((2, 64), _f32)],
    )(s1p, t1, no, b1)


def _u2_body(y1, stats, g1, be1, no, u2):
    i = pl.program_id(0)
    m = _rows_mask(i)
    on = no[:, 0:1]
    u = _bn_apply(y1[...], stats[...], g1[...], be1[...]) * on
    u = jnp.where(m, u, 0.0)
    u2[...] = jnp.concatenate([u, jnp.zeros((RB, 64), _f32)], axis=1)


def _tc_u2(y1, stats, g1, be1, no):
    return pl.pallas_call(
        _u2_body,
        grid=(GN,),
        in_specs=[
            _bspec((RB, 64), lambda i: (i, 0)),
            _bspec((2, 64), lambda i: (0, 0)),
            _bspec((1, 64), lambda i: (0, 0)),
            _bspec((1, 64), lambda i: (0, 0)),
            _bspec((RB, 8), lambda i: (i, 0)),
        ],
        out_specs=[_bspec((RB, 128), lambda i: (i, 0))],
        out_shape=[jax.ShapeDtypeStruct((NP, 128), _f32)],
    )(y1, stats, g1, be1, no)[0]


def _mid2_body(s2p, u2, no, W2, b2, y2, stats, acc):
    i = pl.program_id(0)
    m = _rows_mask(i)
    inn = no[:, 1:2]
    s2 = s2p[0, :, :64] + s2p[1, :, :64] + u2[:, :64]
    y = (s2 * inn) @ W2[...] + b2[...]
    y = jnp.where(m & (y > 0), y, 0.0)
    y2[...] = y
    _acc_stats(i, y, acc, stats)


def _tc_mid2(s2p, u2, no, W2, b2):
    return pl.pallas_call(
        _mid2_body,
        grid=(GN,),
        in_specs=[
            _bspec((2, RB, 128), lambda i: (0, i, 0)),
            _bspec((RB, 128), lambda i: (i, 0)),
            _bspec((RB, 8), lambda i: (i, 0)),
            _bspec((64, 256), lambda i: (0, 0)),
            _bspec((1, 256), lambda i: (0, 0)),
        ],
        out_specs=[
            _bspec((RB, 256), lambda i: (i, 0)),
            _bspec((2, 256), lambda i: (0, 0)),
        ],
        out_shape=[jax.ShapeDtypeStruct((NP, 256), _f32),
                   jax.ShapeDtypeStruct((2, 256), _f32)],
        scratch_shapes=[pltpu.VMEM((2, 256), _f32)],
    )(s2p, u2, no, W2, b2)


def _u3_body(y2, stats, g2, be2, no, u3t):
    i = pl.program_id(0)
    m = _rows_mask(i)
    on = no[:, 0:1]
    u = _bn_apply(y2[...], stats[...], g2[...], be2[...]) * on
    u = jnp.where(m, u, 0.0)
    u3t[...] = jnp.stack([u[:, :128], u[:, 128:]], axis=0)


def _tc_u3(y2, stats, g2, be2, no):
    return pl.pallas_call(
        _u3_body,
        grid=(GN,),
        in_specs=[
            _bspec((RB, 256), lambda i: (i, 0)),
            _bspec((2, 256), lambda i: (0, 0)),
            _bspec((1, 256), lambda i: (0, 0)),
            _bspec((1, 256), lambda i: (0, 0)),
            _bspec((RB, 8), lambda i: (i, 0)),
        ],
        out_specs=[_bspec((2, RB, 128), lambda i: (0, i, 0))],
        out_shape=[jax.ShapeDtypeStruct((2, NP, 128), _f32)],
    )(y2, stats, g2, be2, no)[0]


def _mid3_body(s3pa, s3pb, u3t, no, W3, b3, y3, stats, acc):
    i = pl.program_id(0)
    m = _rows_mask(i)
    inn = no[:, 1:2]
    sa = s3pa[0] + s3pa[1] + u3t[0]
    sb = s3pb[0] + s3pb[1] + u3t[1]
    s3 = jnp.concatenate([sa, sb], axis=1)
    y = (s3 * inn) @ W3[...] + b3[...]
    y = jnp.where(m & (y > 0), y, 0.0)
    y3[...] = y
    _acc_stats(i, y, acc, stats)


def _tc_mid3(s3pa, s3pb, u3t, no, W3, b3):
    return pl.pallas_call(
        _mid3_body,
        grid=(GN,),
        in_specs=[
            _bspec((2, RB, 128), lambda i: (0, i, 0)),
            _bspec((2, RB, 128), lambda i: (0, i, 0)),
            _bspec((2, RB, 128), lambda i: (0, i, 0)),
            _bspec((RB, 8), lambda i: (i, 0)),
            _bspec((256, 512), lambda i: (0, 0)),
            _bspec((1, 512), lambda i: (0, 0)),
        ],
        out_specs=[
            _bspec((RB, 512), lambda i: (i, 0)),
            _bspec((2, 512), lambda i: (0, 0)),
        ],
        out_shape=[jax.ShapeDtypeStruct((NP, 512), _f32),
                   jax.ShapeDtypeStruct((2, 512), _f32)],
        scratch_shapes=[pltpu.VMEM((2, 512), _f32)],
    )(s3pa, s3pb, u3t, no, W3, b3)


def _featres_body(y3, stats, g3, be3, Wgh, Wrh, featf, resf):
    i = pl.program_id(0)
    m = _rows_mask(i)
    h3 = _bn_apply(y3[...], stats[...], g3[...], be3[...])
    h3 = jnp.where(m, h3, 0.0)
    featf[0] = h3 @ Wgh[...]
    resf[0] = h3 @ Wrh[...]


def _tc_featres(y3, stats, g3, be3, Wg, Wres):
    return pl.pallas_call(
        _featres_body,
        grid=(GN, 8),
        in_specs=[
            _bspec((RB, 512), lambda i, h: (i, 0)),
            _bspec((2, 512), lambda i, h: (0, 0)),
            _bspec((1, 512), lambda i, h: (0, 0)),
            _bspec((1, 512), lambda i, h: (0, 0)),
            _bspec((512, 128), lambda i, h: (0, h)),
            _bspec((512, 128), lambda i, h: (0, h)),
        ],
        out_specs=[
            _bspec((1, RB, 128), lambda i, h: (h, i, 0)),
            _bspec((1, RB, 128), lambda i, h: (h, i, 0)),
        ],
        out_shape=[jax.ShapeDtypeStruct((8, NP, 128), _f32),
                   jax.ShapeDtypeStruct((8, NP, 128), _f32)],
    )(y3, stats, g3, be3, Wg, Wres)


def _eler_body(y3, stats, g3, be3, Wg, al, ar, elq, erq, wl):
    i = pl.program_id(0)
    m = _rows_mask(i)
    h3 = _bn_apply(y3[...], stats[...], g3[...], be3[...])
    h3 = jnp.where(m, h3, 0.0)
    Wg3 = Wg[...].reshape(512, 8, 128)
    Al = jnp.sum(Wg3 * al[...][None, :, :], axis=-1)
    Ar = jnp.sum(Wg3 * ar[...][None, :, :], axis=-1)
    el = h3 @ Al
    er = h3 @ Ar
    z120 = jnp.zeros((RB, 120), _f32)
    elq[...] = jnp.concatenate([el, z120], axis=1)
    erq[...] = jnp.concatenate([er, z120], axis=1)
    v = el + er
    v = jnp.where(v > 0, v, 0.2 * v)
    wl[...] = jnp.exp(v)


def _tc_eler(y3, stats, g3, be3, Wg, al, ar):
    return pl.pallas_call(
        _eler_body,
        grid=(GN,),
        in_specs=[
            _bspec((RB, 512), lambda i: (i, 0)),
            _bspec((2, 512), lambda i: (0, 0)),
            _bspec((1, 512), lambda i: (0, 0)),
            _bspec((1, 512), lambda i: (0, 0)),
            _bspec((512, 1024), lambda i: (0, 0)),
            _bspec((8, 128), lambda i: (0, 0)),
            _bspec((8, 128), lambda i: (0, 0)),
        ],
        out_specs=[
            _bspec((RB, 128), lambda i: (i, 0)),
            _bspec((RB, 128), lambda i: (i, 0)),
            _bspec((RB, 8), lambda i: (i, 0)),
        ],
        out_shape=[jax.ShapeDtypeStruct((NP, 128), _f32),
                   jax.ShapeDtypeStruct((NP, 128), _f32),
                   jax.ShapeDtypeStruct((NP, 8), _f32)],
    )(y3, stats, g3, be3, Wg, al, ar)


def _ee_body2(efq, We, ae, eep):
    Ae = jnp.sum(We[...].reshape(2, 8, 128) * ae[...][None, :, :], axis=-1)
    Ae16 = jnp.concatenate([Ae, jnp.zeros((2, 8), _f32)], axis=1)
    rows = []
    for a in range(8):
        parts = []
        if a > 0:
            parts.append(jnp.zeros((2, 16 * a), _f32))
        parts.append(Ae16)
        if a < 7:
            parts.append(jnp.zeros((2, 112 - 16 * a), _f32))
        rows.append(jnp.concatenate(parts, axis=1))
    B = jnp.concatenate(rows, axis=0)
    eep[...] = efq[...] @ B


def _tc_ee(efq, We, ae):
    return pl.pallas_call(
        _ee_body2,
        grid=(8,),
        in_specs=[
            _bspec((EB // 8, 16), lambda i: (i, 0)),
            _bspec((2, 1024), lambda i: (0, 0)),
            _bspec((8, 128), lambda i: (0, 0)),
        ],
        out_specs=[_bspec((EB // 8, 128), lambda i: (i, 0))],
        out_shape=[jax.ShapeDtypeStruct((EPAD // 8, 128), _f32)],
    )(efq, We, ae)[0]


def _final1_body(hofs, rstph, featfh, resfh, denp, wl, bgrh, zp, stats, acc):
    i = pl.program_id(0)
    m = _rows_mask(i)
    cols = []
    for hh in range(4):
        hg = hofs + hh
        wlh = wl[:, hg:hg + 1]
        den = denp[0, :, hg:hg + 1] + denp[1, :, hg:hg + 1] + wlh
        r = (rstph[0, hh] + rstph[1, hh] + featfh[hh] * wlh) / den
        r = r + resfh[hh] + bgrh[hh:hh + 1, :]
        cols.append(r)
    z = jnp.concatenate(cols, axis=1)
    z = jnp.where(m, z, 0.0)
    zp[...] = z
    _acc_stats(i, z, acc, stats)


def _tc_final1h(hofs, rstph, featfh, resfh, denp, wl, bgrh):
    return pl.pallas_call(
        functools.partial(_final1_body, hofs),
        grid=(GN,),
        in_specs=[
            _bspec((2, 4, RB, 128), lambda i: (0, 0, i, 0)),
            _bspec((4, RB, 128), lambda i: (0, i, 0)),
            _bspec((4, RB, 128), lambda i: (0, i, 0)),
            _bspec((2, RB, 128), lambda i: (0, i, 0)),
            _bspec((RB, 8), lambda i: (i, 0)),
            _bspec((4, 128), lambda i: (0, 0)),
        ],
        out_specs=[
            _bspec((RB, 512), lambda i: (i, 0)),
            _bspec((2, 512), lambda i: (0, 0)),
        ],
        out_shape=[jax.ShapeDtypeStruct((NP, 512), _f32),
                   jax.ShapeDtypeStruct((2, 512), _f32)],
        scratch_shapes=[pltpu.VMEM((2, 512), _f32)],
    )(rstph, featfh, resfh, denp, wl, bgrh)


def _final2_body(zpa, zpb, sta, stb, g5, be5, Wa1, ba1, Wa2t, ba2,
                 Wc1, bc1, Wc2t, bc2, a_out, c_out):
    zp = jnp.concatenate([zpa[...], zpb[...]], axis=1)
    stats = jnp.concatenate([sta[...], stb[...]], axis=1)
    z2 = _bn_apply(zp, stats, g5[...], be5[...])
    a1 = jnp.maximum(z2 @ Wa1[...] + ba1[...], 0.0)
    a_out[...] = jnp.sum(a1 * Wa2t[...], axis=1, keepdims=True) + ba2[...]
    c1 = jnp.maximum(z2 @ Wc1[...] + bc1[...], 0.0)
    c_out[...] = jnp.sum(c1 * Wc2t[...], axis=1, keepdims=True) + bc2[...]


def _tc_final2(zpa, zpb, sta, stb, g5, be5, Wa1, ba1, Wa2t, ba2,
               Wc1, bc1, Wc2t, bc2):
    return pl.pallas_call(
        _final2_body,
        grid=(GN,),
        in_specs=[
            _bspec((RB, 512), lambda i: (i, 0)),
            _bspec((RB, 512), lambda i: (i, 0)),
            _bspec((2, 512), lambda i: (0, 0)),
            _bspec((2, 512), lambda i: (0, 0)),
            _bspec((1, 1024), lambda i: (0, 0)),
            _bspec((1, 1024), lambda i: (0, 0)),
            _bspec((1024, 128), lambda i: (0, 0)),
            _bspec((1, 128), lambda i: (0, 0)),
            _bspec((1, 128), lambda i: (0, 0)),
            _bspec((1, 1), lambda i: (0, 0)),
            _bspec((1024, 128), lambda i: (0, 0)),
            _bspec((1, 128), lambda i: (0, 0)),
            _bspec((1, 128), lambda i: (0, 0)),
            _bspec((1, 1), lambda i: (0, 0)),
        ],
        out_specs=[
            _bspec((RB, 1), lambda i: (i, 0)),
            _bspec((RB, 1), lambda i: (i, 0)),
        ],
        out_shape=[jax.ShapeDtypeStruct((N, 1), _f32),
                   jax.ShapeDtypeStruct((N, 1), _f32)],
    )(zpa, zpb, sta, stb, g5, be5, Wa1, ba1, Wa2t, ba2, Wc1, bc1, Wc2t, bc2)


def kernel(x, edge_index, edge_feat, W1, b1, W2, b2, W3, b3,
           g1, be1, g2, be2, g3, be3, g5, be5,
           Wg, We, al, ar, ae, Wres, bg,
           Wa1, ba1, Wa2, ba2, Wc1, bc1, Wc2, bc2):
    i32 = jnp.int32
    r1 = lambda v: v.reshape(1, -1)
    src = edge_index[0]
    dst = edge_index[1]
    padn = EPAD - E
    srcp = jnp.concatenate([src, jnp.full((padn,), DUMMY, i32)])
    dstp = jnp.concatenate([dst, jnp.full((padn,), DUMMY, i32)])
    idxs = jnp.stack([srcp.reshape(NW, NCH, CH),
                      dstp.reshape(NW, NCH, CH)], axis=1)

    ev0 = jnp.zeros((CH, 128), _f32).at[:, 0].set(1.0)
    ev1 = jnp.zeros((CH, 128), _f32).at[:, 1].set(1.0)
    zeros16 = jnp.zeros((NP, 16), _f32)
    zeros128 = jnp.zeros((NP, 128), _f32)
    xp = jnp.pad(x, ((0, NP - N), (0, 0)))
    efq = jnp.pad(edge_feat, ((0, padn), (0, 0))).reshape(EPAD // 8, 16)

    deg = _sc_degrees(idxs, ev0, ev1, zeros128)
    t1, no = _tc_t1(deg, xp, W1)

    s1p = _sc_agg(t1, idxs, zeros128)
    y1, st1 = _tc_post1(s1p, t1, no, r1(b1))
    u2 = _tc_u2(y1, st1, r1(g1), r1(be1), no)

    s2p = _sc_agg(u2, idxs, zeros128)
    y2, st2 = _tc_mid2(s2p, u2, no, W2, r1(b2))
    u3t = _tc_u3(y2, st2, r1(g2), r1(be2), no)

    s3pa = _sc_agg(u3t[0], idxs, zeros128)
    s3pb = _sc_agg(u3t[1], idxs, zeros128)
    y3, st3 = _tc_mid3(s3pa, s3pb, u3t, no, W3, r1(b3))

    featf, resf = _tc_featres(y3, st3, r1(g3), r1(be3), Wg, Wres)
    elq, erq, wl = _tc_eler(y3, st3, r1(g3), r1(be3), Wg, al, ar)
    eep = _tc_ee(efq, We, ae)

    w_hbm, denp = _sc_edge(elq, erq, eep, idxs, zeros128)
    rstp = _sc_gatagg(featf.reshape(8 * NP, 128), w_hbm, idxs, zeros128)

    bgr = bg.reshape(8, 128)
    zpa, sta = _tc_final1h(0, rstp[:, 0:4], featf[0:4], resf[0:4], denp, wl,
                           bgr[0:4])
    zpb, stb = _tc_final1h(4, rstp[:, 4:8], featf[4:8], resf[4:8], denp, wl,
                           bgr[4:8])
    a, c = _tc_final2(zpa, zpb, sta, stb, r1(g5), r1(be5), Wa1, r1(ba1),
                      r1(Wa2[:, 0]), ba2.reshape(1, 1), Wc1, r1(bc1),
                      r1(Wc2[:, 0]), bc2.reshape(1, 1))
    return (a, c)
